# baseline (speedup 1.0000x reference)
import jax
import jax.numpy as jnp
from jax.experimental import pallas as pl
from jax.experimental.pallas import tpu as pltpu

_HT = 1


def _cluster_body(inf_ref, x_ref, w_ref, o_ref, wnt_ref):
    @pl.when(pl.program_id(0) == 0)
    def _prep():
        wv = w_ref[...]
        wn = wv * jax.lax.rsqrt(
            jnp.maximum(jnp.sum(wv * wv, axis=1, keepdims=True), 1e-24))
        wnt_ref[...] = wn.T.astype(jnp.bfloat16)

    b, ht, w, c = x_ref.shape
    xv = x_ref[...].reshape(b * ht * w, c)
    ssq = jnp.sum(xv * xv, axis=1, keepdims=True)
    logits = jnp.dot(xv.astype(jnp.bfloat16), wnt_ref[...],
                     preferred_element_type=jnp.float32)
    ext = jnp.concatenate([logits, ssq], axis=1)
    n1 = ext.shape[1]
    cube = jnp.transpose(ext.reshape(b, ht * w, n1), (2, 1, 0))
    invr = jax.lax.rsqrt(jnp.maximum(cube[n1 - 1:n1], 1e-24))
    o_ref[...] = cube[:n1 - 1] * (invr * inf_ref[0, 0])


def kernel(x, cluster_probe, inference):
    b, c, h, w = x.shape
    n = cluster_probe.shape[0]
    xt = x.transpose(0, 2, 3, 1)
    inf_arr = jnp.asarray(inference, jnp.float32).reshape(1, 1)
    out = pl.pallas_call(
        _cluster_body,
        grid=(h // _HT,),
        in_specs=[
            pl.BlockSpec(memory_space=pltpu.SMEM),
            pl.BlockSpec((b, _HT, w, c), lambda i: (0, i, 0, 0)),
            pl.BlockSpec((n, c), lambda i: (0, 0)),
        ],
        out_specs=pl.BlockSpec((n, _HT * w, b), lambda i: (0, i, 0)),
        out_shape=jax.ShapeDtypeStruct((n, h * w, b), jnp.float32),
        scratch_shapes=[pltpu.VMEM((c, n), jnp.bfloat16)],
        compiler_params=pltpu.CompilerParams(
            vmem_limit_bytes=60 * 1024 * 1024),
    )(inf_arr, xt, cluster_probe)
    return out.reshape(n, h, w, b).transpose(3, 0, 1, 2)


# speedup vs baseline: 1.2175x; 1.2175x over previous
import jax
import jax.numpy as jnp
from jax.experimental import pallas as pl
from jax.experimental.pallas import tpu as pltpu

_HT = 2


def _cluster_body(inf_ref, x_ref, w_ref, o_ref, wnt_ref):
    @pl.when(pl.program_id(0) == 0)
    def _prep():
        wv = w_ref[...]
        wn = wv * jax.lax.rsqrt(
            jnp.maximum(jnp.sum(wv * wv, axis=1, keepdims=True), 1e-24))
        wnt_ref[...] = wn.T.astype(jnp.bfloat16)

    b, ht, w, c = x_ref.shape
    xv = x_ref[...].reshape(b * ht * w, c)
    ssq = jnp.sum(xv * xv, axis=1, keepdims=True)
    logits = jnp.dot(xv.astype(jnp.bfloat16), wnt_ref[...],
                     preferred_element_type=jnp.float32)
    ext = jnp.concatenate([logits, ssq], axis=1)
    n1 = ext.shape[1]
    cube = jnp.transpose(ext.reshape(b, ht * w, n1), (2, 1, 0))
    invr = jax.lax.rsqrt(jnp.maximum(cube[n1 - 1:n1], 1e-24))
    o_ref[...] = cube[:n1 - 1] * (invr * inf_ref[0, 0])


def kernel(x, cluster_probe, inference):
    b, c, h, w = x.shape
    n = cluster_probe.shape[0]
    xt = x.transpose(0, 2, 3, 1)
    inf_arr = jnp.asarray(inference, jnp.float32).reshape(1, 1)
    out = pl.pallas_call(
        _cluster_body,
        grid=(h // _HT,),
        in_specs=[
            pl.BlockSpec(memory_space=pltpu.SMEM),
            pl.BlockSpec((b, _HT, w, c), lambda i: (0, i, 0, 0)),
            pl.BlockSpec((n, c), lambda i: (0, 0)),
        ],
        out_specs=pl.BlockSpec((n, _HT * w, b), lambda i: (0, i, 0)),
        out_shape=jax.ShapeDtypeStruct((n, h * w, b), jnp.float32),
        scratch_shapes=[pltpu.VMEM((c, n), jnp.bfloat16)],
        compiler_params=pltpu.CompilerParams(
            vmem_limit_bytes=60 * 1024 * 1024),
    )(inf_arr, xt, cluster_probe)
    return out.reshape(n, h, w, b).transpose(3, 0, 1, 2)
